# baseline (speedup 1.0000x reference)
import functools

import jax
import jax.numpy as jnp
from jax import lax
from jax.experimental import pallas as pl
from jax.experimental.pallas import tpu as pltpu
from jax.experimental.pallas import tpu_sc as plsc

_B = 4096
_F = 26
_V = 100000
_D = 16
_H = 128

_NC = 2
_NS = 16
_NW = _NC * _NS
_SPT = _B // _NW
_VB = _SPT // 16


def _sc_body(idxt_hbm, xvt_hbm, sotab_hbm, fotab_hbm,
             so2_hbm, fosum_hbm,
             idx_v, xvt_v, val_v, fo_v, so2_v, fos_v,
             sem_in, sem_g, sem_f, sem_out):
    cid = lax.axis_index("c")
    sid = lax.axis_index("s")
    w = sid * _NC + cid

    cps = [
        pltpu.async_copy(idxt_hbm.at[w], idx_v, sem_in),
        pltpu.async_copy(xvt_hbm.at[w], xvt_v, sem_in),
    ]
    for cp in cps:
        cp.wait()

    gcps = []
    fcps = []
    for f in range(_F):
        for d in range(_D):
            r = f * _D + d
            gcps.append(pltpu.async_copy(
                sotab_hbm.at[r].at[idx_v.at[f]],
                val_v.at[pl.ds(r * _SPT, _SPT)],
                sem_g,
            ))
        fcps.append(pltpu.async_copy(
            fotab_hbm.at[f].at[idx_v.at[f]],
            fo_v.at[pl.ds(f * _SPT, _SPT)],
            sem_f,
        ))
    for cp in gcps:
        cp.wait()
    for cp in fcps:
        cp.wait()

    iota16 = lax.broadcasted_iota(jnp.int32, (16,), 0)

    def vb_body(vb, _):
        off = vb * 16
        s_acc = [jnp.zeros((16,), jnp.float32) for _ in range(_D)]
        q_acc = [jnp.zeros((16,), jnp.float32) for _ in range(_D)]
        fo_acc = jnp.zeros((16,), jnp.float32)
        for f in range(_F):
            xv = xvt_v[pl.ds(f * _SPT + off, 16)]
            for d in range(_D):
                e = val_v[pl.ds((f * _D + d) * _SPT + off, 16)] * xv
                s_acc[d] = s_acc[d] + e
                q_acc[d] = q_acc[d] + e * e
            fov = plsc.load_gather(
                fo_v, [jnp.full((16,), f * _SPT + off, jnp.int32) + iota16,
                       jnp.zeros((16,), jnp.int32)])
            fo_acc = fo_acc + fov * xv
        for d in range(_D):
            so2_v[pl.ds(d * _SPT + off, 16)] = (
                s_acc[d] * s_acc[d] - q_acc[d]) * 0.5
        fos_v[pl.ds(off, 16)] = fo_acc
        return _

    lax.fori_loop(0, _VB, vb_body, 0)

    out_cps = [
        pltpu.async_copy(so2_v, so2_hbm.at[w], sem_out),
        pltpu.async_copy(fos_v, fosum_hbm.at[w], sem_out),
    ]
    for cp in out_cps:
        cp.wait()


@functools.partial(
    pl.kernel,
    out_type=(
        jax.ShapeDtypeStruct((_NW, _D * _SPT), jnp.float32),
        jax.ShapeDtypeStruct((_NW, _SPT), jnp.float32),
    ),
    mesh=plsc.VectorSubcoreMesh(core_axis_name="c", subcore_axis_name="s"),
    compiler_params=pltpu.CompilerParams(
        use_tc_tiling_on_sc=False, needs_layout_passes=False),
    scratch_types=(
        pltpu.VMEM((_F, _SPT), jnp.int32),
        pltpu.VMEM((_F * _SPT,), jnp.float32),
        pltpu.VMEM((_F * _D * _SPT,), jnp.float32),
        pltpu.VMEM((_F * _SPT, 1), jnp.float32),
        pltpu.VMEM((_D * _SPT,), jnp.float32),
        pltpu.VMEM((_SPT,), jnp.float32),
        pltpu.SemaphoreType.DMA,
        pltpu.SemaphoreType.DMA,
        pltpu.SemaphoreType.DMA,
        pltpu.SemaphoreType.DMA,
    ),
)
def _sc_gather_fm(idxt, xvt, sotab, fotab, so2, fosum, *rest):
    _sc_body(idxt, xvt, sotab, fotab, so2, fosum, *rest)


def _tc_body(so3_ref, fos_ref, w0_ref, b0_ref, w1_ref, b1_ref, bias_ref,
             out_ref):
    x = so3_ref[...].transpose(0, 2, 1).reshape(_B, _D)
    h = jnp.dot(x, w0_ref[...], preferred_element_type=jnp.float32)
    h = jnp.maximum(h + b0_ref[...], 0.0)
    h = jnp.dot(h, w1_ref[...], preferred_element_type=jnp.float32)
    h = jnp.maximum(h + b1_ref[...], 0.0)
    hsum = jnp.sum(h.reshape(_NW, _SPT, _H), axis=2)
    out_ref[...] = bias_ref[0, 0] + fos_ref[...] + hsum


def _tc_mlp(so3, fosum, W0, b0, W1, b1, bias2d):
    return pl.pallas_call(
        _tc_body,
        out_shape=jax.ShapeDtypeStruct((_NW, _SPT), jnp.float32),
        in_specs=[
            pl.BlockSpec(memory_space=pltpu.VMEM),
            pl.BlockSpec(memory_space=pltpu.VMEM),
            pl.BlockSpec(memory_space=pltpu.VMEM),
            pl.BlockSpec(memory_space=pltpu.VMEM),
            pl.BlockSpec(memory_space=pltpu.VMEM),
            pl.BlockSpec(memory_space=pltpu.VMEM),
            pl.BlockSpec(memory_space=pltpu.SMEM),
        ],
        out_specs=pl.BlockSpec(memory_space=pltpu.VMEM),
    )(so3, fosum, W0, b0, W1, b1, bias2d)


def kernel(Xi, Xv, fo_emb, so_emb, W0, b0, W1, b1, b):
    idx = Xi[:, :, 0].astype(jnp.int32)
    idxt = idx.reshape(_NW, _SPT, _F).transpose(0, 2, 1)
    xvt = Xv.reshape(_NW, _SPT, _F).transpose(0, 2, 1).reshape(_NW, _F * _SPT)
    sotab = so_emb.transpose(0, 2, 1).reshape(_F * _D, _V)
    fotab = fo_emb
    so2, fosum = _sc_gather_fm(idxt, xvt, sotab, fotab)
    so3 = so2.reshape(_NW, _D, _SPT)
    out2d = _tc_mlp(so3, fosum, W0, b0, W1, b1,
                    jnp.reshape(b.astype(jnp.float32), (1, 1)))
    return out2d.reshape(_B)


# speedup vs baseline: 9.4330x; 9.4330x over previous
import functools

import jax
import jax.numpy as jnp
from jax import lax
from jax.experimental import pallas as pl
from jax.experimental.pallas import tpu as pltpu
from jax.experimental.pallas import tpu_sc as plsc

_B = 4096
_F = 26
_V = 100000
_D = 16
_H = 128

_NC = 2
_NS = 16
_NW = _NC * _NS
_SPT = _B // _NW
_VB = _SPT // 16


def _sc_body(idxt_hbm, xvt_hbm, sotab_hbm, fotab_hbm,
             so2_hbm, fosum_hbm,
             idx_v, xvt_v, val_v, fo_v, so2_v, fos_v,
             sem_in, sem_g, sem_f, sem_out):
    cid = lax.axis_index("c")
    sid = lax.axis_index("s")
    w = sid * _NC + cid

    cps = [
        pltpu.async_copy(idxt_hbm.at[w], idx_v, sem_in),
        pltpu.async_copy(xvt_hbm.at[w], xvt_v, sem_in),
    ]
    for cp in cps:
        cp.wait()

    gcps = []
    fcps = []
    for f in range(_F):
        for d in range(_D):
            r = f * _D + d
            gcps.append(pltpu.async_copy(
                sotab_hbm.at[r].at[idx_v.at[f]],
                val_v.at[pl.ds(r * _SPT, _SPT)],
                sem_g,
            ))
        fcps.append(pltpu.async_copy(
            fotab_hbm.at[f].at[idx_v.at[f]],
            fo_v.at[pl.ds(f * _SPT, _SPT)],
            sem_f,
        ))
    for cp in gcps:
        cp.wait()
    for cp in fcps:
        cp.wait()

    def vb_body(vb, _):
        off = vb * 16
        s_acc = [jnp.zeros((16,), jnp.float32) for _ in range(_D)]
        q_acc = [jnp.zeros((16,), jnp.float32) for _ in range(_D)]
        fo_acc = jnp.zeros((16,), jnp.float32)
        for f in range(_F):
            xv = xvt_v[pl.ds(f * _SPT + off, 16)]
            for d in range(_D):
                e = val_v[pl.ds((f * _D + d) * _SPT + off, 16)] * xv
                s_acc[d] = s_acc[d] + e
                q_acc[d] = q_acc[d] + e * e
            fo_acc = fo_acc + fo_v[pl.ds(f * _SPT + off, 16)] * xv
        for d in range(_D):
            so2_v[pl.ds(d * _SPT + off, 16)] = (
                s_acc[d] * s_acc[d] - q_acc[d]) * 0.5
        fos_v[pl.ds(off, 16)] = fo_acc
        return _

    lax.fori_loop(0, _VB, vb_body, 0)

    out_cps = [
        pltpu.async_copy(so2_v, so2_hbm.at[w], sem_out),
        pltpu.async_copy(fos_v, fosum_hbm.at[w], sem_out),
    ]
    for cp in out_cps:
        cp.wait()


@functools.partial(
    pl.kernel,
    out_type=(
        jax.ShapeDtypeStruct((_NW, _D * _SPT), jnp.float32),
        jax.ShapeDtypeStruct((_NW, _SPT), jnp.float32),
    ),
    mesh=plsc.VectorSubcoreMesh(core_axis_name="c", subcore_axis_name="s"),
    compiler_params=pltpu.CompilerParams(use_tc_tiling_on_sc=False),
    scratch_types=(
        pltpu.VMEM((_F, _SPT), jnp.int32),
        pltpu.VMEM((_F * _SPT,), jnp.float32),
        pltpu.VMEM((_F * _D * _SPT,), jnp.float32),
        pltpu.VMEM((_F * _SPT,), jnp.float32),
        pltpu.VMEM((_D * _SPT,), jnp.float32),
        pltpu.VMEM((_SPT,), jnp.float32),
        pltpu.SemaphoreType.DMA,
        pltpu.SemaphoreType.DMA,
        pltpu.SemaphoreType.DMA,
        pltpu.SemaphoreType.DMA,
    ),
)
def _sc_gather_fm(idxt, xvt, sotab, fotab, so2, fosum, *rest):
    _sc_body(idxt, xvt, sotab, fotab, so2, fosum, *rest)


def _tc_body(so3_ref, fos_ref, w0_ref, b0_ref, w1_ref, b1_ref, bias_ref,
             out_ref):
    x = so3_ref[...].transpose(0, 2, 1).reshape(_B, _D)
    h = jnp.dot(x, w0_ref[...], preferred_element_type=jnp.float32)
    h = jnp.maximum(h + b0_ref[...], 0.0)
    h = jnp.dot(h, w1_ref[...], preferred_element_type=jnp.float32)
    h = jnp.maximum(h + b1_ref[...], 0.0)
    hsum = jnp.sum(h.reshape(_NW, _SPT, _H), axis=2)
    out_ref[...] = bias_ref[0, 0] + fos_ref[...] + hsum


def _tc_mlp(so3, fosum, W0, b0, W1, b1, bias2d):
    return pl.pallas_call(
        _tc_body,
        out_shape=jax.ShapeDtypeStruct((_NW, _SPT), jnp.float32),
        in_specs=[
            pl.BlockSpec(memory_space=pltpu.VMEM),
            pl.BlockSpec(memory_space=pltpu.VMEM),
            pl.BlockSpec(memory_space=pltpu.VMEM),
            pl.BlockSpec(memory_space=pltpu.VMEM),
            pl.BlockSpec(memory_space=pltpu.VMEM),
            pl.BlockSpec(memory_space=pltpu.VMEM),
            pl.BlockSpec(memory_space=pltpu.SMEM),
        ],
        out_specs=pl.BlockSpec(memory_space=pltpu.VMEM),
    )(so3, fosum, W0, b0, W1, b1, bias2d)


def kernel(Xi, Xv, fo_emb, so_emb, W0, b0, W1, b1, b):
    idx = Xi[:, :, 0].astype(jnp.int32)
    idxt = idx.reshape(_NW, _SPT, _F).transpose(0, 2, 1)
    xvt = Xv.reshape(_NW, _SPT, _F).transpose(0, 2, 1).reshape(_NW, _F * _SPT)
    sotab = so_emb.transpose(0, 2, 1).reshape(_F * _D, _V)
    fotab = fo_emb.reshape(_F, _V)
    so2, fosum = _sc_gather_fm(idxt, xvt, sotab, fotab)
    so3 = so2.reshape(_NW, _D, _SPT)
    out2d = _tc_mlp(so3, fosum, W0, b0, W1, b1,
                    jnp.reshape(b.astype(jnp.float32), (1, 1)))
    return out2d.reshape(_B)
